# Initial kernel scaffold; baseline (speedup 1.0000x reference)
import jax
import jax.numpy as jnp
from jax.experimental import pallas as pl


def kernel(x, edge_index, W1, b1, W2, b2, W3, b3, W4, b4, W5, b5):
    raise NotImplementedError("write your pallas kernel here")


# speedup vs baseline: 5.8406x; 5.8406x over previous
import functools

import jax
import jax.numpy as jnp
from jax import lax
from jax.experimental import pallas as pl
from jax.experimental.pallas import tpu as pltpu
from jax.experimental.pallas import tpu_sc as plsc

N = 10000
D = 256
H = 128
CHUNK = 128
NSUB = 16
ACC_ROWS = 10240
NPT = 624
BN = 1000


def _ceil_to(v, m):
    return -(-v // m) * m


@functools.cache
def _mesh():
    return plsc.VectorSubcoreMesh(core_axis_name="c", subcore_axis_name="s",
                                  num_cores=2, num_subcores=NSUB)


def _copy_out(acc, out, s):
    @pl.when(s < NSUB - 1)
    def _():
        o = pl.multiple_of(s * NPT, 8)
        pltpu.sync_copy(acc.at[pl.ds(o, NPT)], out.at[pl.ds(o, NPT)])

    @pl.when(s == NSUB - 1)
    def _():
        last = N - (NSUB - 1) * NPT
        pltpu.sync_copy(acc.at[pl.ds((NSUB - 1) * NPT, last)],
                        out.at[pl.ds((NSUB - 1) * NPT, last)])


def _deg_body(dst_hbm, ones_hbm, z_hbm, out0, out1, accd, idx_d, ones_v):
    c = lax.axis_index("c")
    s = lax.axis_index("s")
    w = c * NSUB + s
    zr = ACC_ROWS // NSUB
    pltpu.sync_copy(z_hbm, accd.at[pl.ds(pl.multiple_of(s * zr, 8), zr)])
    pltpu.sync_copy(ones_hbm, ones_v)
    pltpu.sync_copy(dst_hbm.at[w], idx_d)
    plsc.subcore_barrier()
    nchunks = idx_d.shape[0]

    def step(j, carry):
        pltpu.sync_copy(ones_v, accd.at[idx_d.at[j]], add=True)
        return carry

    lax.fori_loop(0, nchunks, step, 0)
    plsc.subcore_barrier()

    @pl.when(c == 0)
    def _():
        _copy_out(accd, out0, s)

    @pl.when(c == 1)
    def _():
        _copy_out(accd, out1, s)


WIN = 8


def _agg_body(hs0, hs1, src_hbm, dst_hbm, z_hbm, out0, out1,
              acc, idx_s, idx_d, buf0, buf1, sem0, sem1):
    c = lax.axis_index("c")
    s = lax.axis_index("s")
    zr = ACC_ROWS // NSUB
    pltpu.sync_copy(z_hbm, acc.at[pl.ds(pl.multiple_of(s * zr, 8), zr)])
    plsc.subcore_barrier()
    nchunks = src_hbm.shape[1]

    def run(hs):
        def window(w, carry):
            wo = pl.multiple_of(w * WIN, 8)
            pltpu.sync_copy(src_hbm.at[s, pl.ds(wo, WIN)], idx_s)
            pltpu.sync_copy(dst_hbm.at[s, pl.ds(wo, WIN)], idx_d)
            pltpu.async_copy(hs.at[idx_s.at[0]], buf0, sem0).wait()
            for j in range(0, WIN, 2):
                g1 = pltpu.async_copy(hs.at[idx_s.at[j + 1]], buf1, sem1)
                pltpu.sync_copy(buf0, acc.at[idx_d.at[j]], add=True)
                g1.wait()
                if j + 2 < WIN:
                    pltpu.async_copy(hs.at[idx_s.at[j + 2]], buf0, sem0).wait()
                pltpu.sync_copy(buf1, acc.at[idx_d.at[j + 1]], add=True)
            return carry

        lax.fori_loop(0, nchunks // WIN, window, 0)

    @pl.when(c == 0)
    def _():
        run(hs0)

    @pl.when(c == 1)
    def _():
        run(hs1)

    plsc.subcore_barrier()

    @pl.when(c == 0)
    def _():
        _copy_out(acc, out0, s)

    @pl.when(c == 1)
    def _():
        _copy_out(acc, out1, s)


def _make_deg_call(nchunks):
    return pl.kernel(
        _deg_body,
        out_type=[jax.ShapeDtypeStruct((N, 16), jnp.float32),
                  jax.ShapeDtypeStruct((N, 16), jnp.float32)],
        mesh=_mesh(),
        scratch_types=[
            pltpu.VMEM_SHARED((ACC_ROWS, 16), jnp.float32),
            pltpu.VMEM((nchunks, CHUNK), jnp.int32),
            pltpu.VMEM((CHUNK, 16), jnp.float32),
        ],
    )


def _make_agg_call(nchunks):
    return pl.kernel(
        _agg_body,
        out_type=[jax.ShapeDtypeStruct((N, H), jnp.float32),
                  jax.ShapeDtypeStruct((N, H), jnp.float32)],
        mesh=_mesh(),
        scratch_types=[
            pltpu.VMEM_SHARED((ACC_ROWS, H), jnp.float32),
            pltpu.VMEM((WIN, CHUNK), jnp.int32),
            pltpu.VMEM((WIN, CHUNK), jnp.int32),
            pltpu.VMEM((CHUNK, H), jnp.float32),
            pltpu.VMEM((CHUNK, H), jnp.float32),
            pltpu.SemaphoreType.DMA,
            pltpu.SemaphoreType.DMA,
        ],
    )


def _l1_body(x_ref, w_ref, d0_ref, d1_ref, hs0_ref, hs1_ref, dinv_ref):
    deg = d0_ref[:, :1] + d1_ref[:, :1] + 1.0
    dinv = lax.rsqrt(deg)
    h = jnp.dot(x_ref[...], w_ref[...], preferred_element_type=jnp.float32)
    hs = h * dinv
    hs0_ref[...] = hs[:, :H]
    hs1_ref[...] = hs[:, H:]
    dinv_ref[...] = jnp.broadcast_to(dinv, (BN, H))


def _mid_body(a0_ref, a1_ref, h0_ref, h1_ref, dinv_ref, b_ref, w_ref,
              o0_ref, o1_ref):
    dinv = dinv_ref[...]
    a0 = jnp.maximum(dinv * (a0_ref[...] + h0_ref[...]) + b_ref[0:1, :], 0.0)
    a1 = jnp.maximum(dinv * (a1_ref[...] + h1_ref[...]) + b_ref[1:2, :], 0.0)
    a = jnp.concatenate([a0, a1], axis=1)
    h = jnp.dot(a, w_ref[...], preferred_element_type=jnp.float32)
    hs = h * dinv[:, :1]
    o0_ref[...] = hs[:, :H]
    o1_ref[...] = hs[:, H:]


def _fin_body(a0_ref, a1_ref, h0_ref, h1_ref, dinv_ref, b_ref, out_ref):
    dinv = dinv_ref[...]
    v0 = dinv * (a0_ref[...] + h0_ref[...]) + b_ref[0:1, :]
    v1 = dinv * (a1_ref[...] + h1_ref[...]) + b_ref[1:2, :]
    v = jnp.concatenate([v0, v1], axis=1)
    m = jnp.max(v, axis=1, keepdims=True)
    lse = jnp.log(jnp.sum(jnp.exp(v - m), axis=1, keepdims=True)) + m
    out_ref[...] = v - lse


def _rows(shape):
    return pl.BlockSpec(shape, lambda i: (i,) + (0,) * (len(shape) - 1))


def _full(shape):
    return pl.BlockSpec(shape, lambda i: (0,) * len(shape))


_GRID = N // BN

_l1_call = pl.pallas_call(
    _l1_body,
    grid=(_GRID,),
    in_specs=[_rows((BN, D)), _full((D, D)), _rows((BN, 16)), _rows((BN, 16))],
    out_specs=[_rows((BN, H)), _rows((BN, H)), _rows((BN, H))],
    out_shape=[jax.ShapeDtypeStruct((N, H), jnp.float32),
               jax.ShapeDtypeStruct((N, H), jnp.float32),
               jax.ShapeDtypeStruct((N, H), jnp.float32)],
)

_mid_call = pl.pallas_call(
    _mid_body,
    grid=(_GRID,),
    in_specs=[_rows((BN, H))] * 5 + [_full((2, H)), _full((D, D))],
    out_specs=[_rows((BN, H)), _rows((BN, H))],
    out_shape=[jax.ShapeDtypeStruct((N, H), jnp.float32),
               jax.ShapeDtypeStruct((N, H), jnp.float32)],
)

_fin_call = pl.pallas_call(
    _fin_body,
    grid=(_GRID,),
    in_specs=[_rows((BN, H))] * 5 + [_full((2, H))],
    out_specs=_rows((BN, D)),
    out_shape=jax.ShapeDtypeStruct((N, D), jnp.float32),
)


def kernel(x, edge_index, W1, b1, W2, b2, W3, b3, W4, b4, W5, b5):
    src = edge_index[0]
    dst = edge_index[1]
    e = src.shape[0]

    agg_chunks = _ceil_to(_ceil_to(e, NSUB * CHUNK) // (NSUB * CHUNK), WIN)
    e_agg = NSUB * CHUNK * agg_chunks
    deg_chunks = _ceil_to(e, 2 * NSUB * CHUNK) // (2 * NSUB * CHUNK)
    e_deg = 2 * NSUB * CHUNK * deg_chunks

    i32 = jnp.int32
    src_a = jnp.concatenate([src, jnp.zeros((e_agg - e,), i32)])
    src_a = src_a.reshape(NSUB, agg_chunks, CHUNK)
    dst_a = jnp.concatenate([dst, jnp.full((e_agg - e,), N, i32)])
    dst_a = dst_a.reshape(NSUB, agg_chunks, CHUNK)
    dst_d = jnp.concatenate([dst, jnp.full((e_deg - e,), N, i32)])
    dst_d = dst_d.reshape(2 * NSUB, deg_chunks, CHUNK)

    zrows = jnp.zeros((ACC_ROWS // NSUB, H), jnp.float32)
    zdeg = jnp.zeros((ACC_ROWS // NSUB, 16), jnp.float32)
    ones16 = jnp.ones((CHUNK, 16), jnp.float32)

    deg0, deg1 = _make_deg_call(deg_chunks)(dst_d, ones16, zdeg)
    hs0, hs1, dinv = _l1_call(x, W1, deg0, deg1)

    agg = _make_agg_call(agg_chunks)
    for b_prev, W in ((b1, W2), (b2, W3), (b3, W4), (b4, W5)):
        acc0, acc1 = agg(hs0, hs1, src_a, dst_a, zrows)
        hs0, hs1 = _mid_call(acc0, acc1, hs0, hs1, dinv,
                             b_prev.reshape(2, H), W)
    acc0, acc1 = agg(hs0, hs1, src_a, dst_a, zrows)
    return _fin_call(acc0, acc1, hs0, hs1, dinv, b5.reshape(2, H))


# speedup vs baseline: 6.2968x; 1.0781x over previous
import functools

import jax
import jax.numpy as jnp
from jax import lax
from jax.experimental import pallas as pl
from jax.experimental.pallas import tpu as pltpu
from jax.experimental.pallas import tpu_sc as plsc

N = 10000
D = 256
H = 128
CHUNK = 128
NSUB = 16
ACC_ROWS = 10240
NPT = 624
BN = 1000


def _ceil_to(v, m):
    return -(-v // m) * m


@functools.cache
def _mesh():
    return plsc.VectorSubcoreMesh(core_axis_name="c", subcore_axis_name="s",
                                  num_cores=2, num_subcores=NSUB)


def _copy_out(acc, out, s):
    @pl.when(s < NSUB - 1)
    def _():
        o = pl.multiple_of(s * NPT, 8)
        pltpu.sync_copy(acc.at[pl.ds(o, NPT)], out.at[pl.ds(o, NPT)])

    @pl.when(s == NSUB - 1)
    def _():
        last = N - (NSUB - 1) * NPT
        pltpu.sync_copy(acc.at[pl.ds((NSUB - 1) * NPT, last)],
                        out.at[pl.ds((NSUB - 1) * NPT, last)])


def _deg_body(dst_hbm, ones_hbm, z_hbm, out0, out1, accd, idx_d, ones_v):
    c = lax.axis_index("c")
    s = lax.axis_index("s")
    w = c * NSUB + s
    zr = ACC_ROWS // NSUB
    pltpu.sync_copy(z_hbm, accd.at[pl.ds(pl.multiple_of(s * zr, 8), zr)])
    pltpu.sync_copy(ones_hbm, ones_v)
    plsc.subcore_barrier()
    nchunks = dst_hbm.shape[1]

    def window(wi, carry):
        wo = pl.multiple_of(wi * WINC, 8)
        pltpu.sync_copy(dst_hbm.at[w, pl.ds(wo, WINC)], idx_d)
        for j in range(WINC):
            pltpu.sync_copy(ones_v, accd.at[idx_d.at[j]], add=True)
        return carry

    lax.fori_loop(0, nchunks // WINC, window, 0)
    plsc.subcore_barrier()

    @pl.when(c == 0)
    def _():
        _copy_out(accd, out0, s)

    @pl.when(c == 1)
    def _():
        _copy_out(accd, out1, s)


GCH = 128
NBUF = 2
WINC = 8


def _agg_body(hs0, hs1, src_hbm, dst_hbm, z_hbm, out0, out1,
              acc, idx_s, idx_d, buf0, buf1,
              gsem0, gsem1, ssem0, ssem1):
    bufs = (buf0, buf1)
    gsems = (gsem0, gsem1)
    ssems = (ssem0, ssem1)
    c = lax.axis_index("c")
    s = lax.axis_index("s")
    zr = ACC_ROWS // NSUB
    pltpu.sync_copy(z_hbm, acc.at[pl.ds(pl.multiple_of(s * zr, 8), zr)])
    plsc.subcore_barrier()
    nchunks = src_hbm.shape[1]

    def run(hs):
        def window(w, carry):
            wo = pl.multiple_of(w * WINC, 8)
            pltpu.sync_copy(src_hbm.at[s, pl.ds(wo, WINC)], idx_s)
            pltpu.sync_copy(dst_hbm.at[s, pl.ds(wo, WINC)], idx_d)
            g = pltpu.async_copy(hs.at[idx_s.at[0]], bufs[0], gsems[0])
            for j in range(WINC):
                b = j % 2
                gn = None
                if j + 1 < WINC:
                    gn = pltpu.async_copy(hs.at[idx_s.at[j + 1]],
                                          bufs[1 - b], gsems[1 - b])
                g.wait()
                pltpu.sync_copy(bufs[b], acc.at[idx_d.at[j]], add=True)
                g = gn
            return carry

        lax.fori_loop(0, nchunks // WINC, window, 0)

    @pl.when(c == 0)
    def _():
        run(hs0)

    @pl.when(c == 1)
    def _():
        run(hs1)

    plsc.subcore_barrier()

    @pl.when(c == 0)
    def _():
        _copy_out(acc, out0, s)

    @pl.when(c == 1)
    def _():
        _copy_out(acc, out1, s)


def _make_deg_call(nchunks):
    return pl.kernel(
        _deg_body,
        out_type=[jax.ShapeDtypeStruct((N, H), jnp.float32),
                  jax.ShapeDtypeStruct((N, H), jnp.float32)],
        mesh=_mesh(),
        scratch_types=[
            pltpu.VMEM_SHARED((ACC_ROWS, H), jnp.float32),
            pltpu.VMEM((WINC, CHUNK), jnp.int32),
            pltpu.VMEM((CHUNK, H), jnp.float32),
        ],
    )


def _make_agg_call(nchunks):
    return pl.kernel(
        _agg_body,
        out_type=[jax.ShapeDtypeStruct((N, H), jnp.float32),
                  jax.ShapeDtypeStruct((N, H), jnp.float32)],
        mesh=_mesh(),
        scratch_types=[
            pltpu.VMEM_SHARED((ACC_ROWS, H), jnp.float32),
            pltpu.VMEM((WINC, GCH), jnp.int32),
            pltpu.VMEM((WINC, GCH), jnp.int32),
            pltpu.VMEM((GCH, H), jnp.float32),
            pltpu.VMEM((GCH, H), jnp.float32),
        ] + [pltpu.SemaphoreType.DMA] * 4,
    )


def _l1_body(x_ref, w_ref, d0_ref, d1_ref, hs0_ref, hs1_ref, dinv_ref):
    deg = d0_ref[:, :1] + d1_ref[:, :1] + 1.0
    dinv = lax.rsqrt(deg)
    h = jnp.dot(x_ref[...], w_ref[...], preferred_element_type=jnp.float32)
    hs = h * dinv
    hs0_ref[...] = hs[:, :H]
    hs1_ref[...] = hs[:, H:]
    dinv_ref[...] = jnp.broadcast_to(dinv, (BN, H))


def _mid_body(a0_ref, a1_ref, h0_ref, h1_ref, dinv_ref, b_ref, w_ref,
              o0_ref, o1_ref):
    dinv = dinv_ref[...]
    a0 = jnp.maximum(dinv * (a0_ref[...] + h0_ref[...]) + b_ref[0:1, :], 0.0)
    a1 = jnp.maximum(dinv * (a1_ref[...] + h1_ref[...]) + b_ref[1:2, :], 0.0)
    a = jnp.concatenate([a0, a1], axis=1)
    h = jnp.dot(a, w_ref[...], preferred_element_type=jnp.float32)
    hs = h * dinv[:, :1]
    o0_ref[...] = hs[:, :H]
    o1_ref[...] = hs[:, H:]


def _fin_body(a0_ref, a1_ref, h0_ref, h1_ref, dinv_ref, b_ref, out_ref):
    dinv = dinv_ref[...]
    v0 = dinv * (a0_ref[...] + h0_ref[...]) + b_ref[0:1, :]
    v1 = dinv * (a1_ref[...] + h1_ref[...]) + b_ref[1:2, :]
    v = jnp.concatenate([v0, v1], axis=1)
    m = jnp.max(v, axis=1, keepdims=True)
    lse = jnp.log(jnp.sum(jnp.exp(v - m), axis=1, keepdims=True)) + m
    out_ref[...] = v - lse


def _rows(shape):
    return pl.BlockSpec(shape, lambda i: (i,) + (0,) * (len(shape) - 1))


def _full(shape):
    return pl.BlockSpec(shape, lambda i: (0,) * len(shape))


_GRID = N // BN

_l1_call = pl.pallas_call(
    _l1_body,
    grid=(_GRID,),
    in_specs=[_rows((BN, D)), _full((D, D)), _rows((BN, H)), _rows((BN, H))],
    out_specs=[_rows((BN, H)), _rows((BN, H)), _rows((BN, H))],
    out_shape=[jax.ShapeDtypeStruct((N, H), jnp.float32),
               jax.ShapeDtypeStruct((N, H), jnp.float32),
               jax.ShapeDtypeStruct((N, H), jnp.float32)],
)

_mid_call = pl.pallas_call(
    _mid_body,
    grid=(_GRID,),
    in_specs=[_rows((BN, H))] * 5 + [_full((2, H)), _full((D, D))],
    out_specs=[_rows((BN, H)), _rows((BN, H))],
    out_shape=[jax.ShapeDtypeStruct((N, H), jnp.float32),
               jax.ShapeDtypeStruct((N, H), jnp.float32)],
)

_fin_call = pl.pallas_call(
    _fin_body,
    grid=(_GRID,),
    in_specs=[_rows((BN, H))] * 5 + [_full((2, H))],
    out_specs=_rows((BN, D)),
    out_shape=jax.ShapeDtypeStruct((N, D), jnp.float32),
)


def kernel(x, edge_index, W1, b1, W2, b2, W3, b3, W4, b4, W5, b5):
    src = edge_index[0]
    dst = edge_index[1]
    e = src.shape[0]

    agg_chunks = _ceil_to(_ceil_to(e, NSUB * GCH) // (NSUB * GCH), WINC)
    e_agg = NSUB * GCH * agg_chunks
    deg_chunks = _ceil_to(_ceil_to(e, 2 * NSUB * CHUNK) // (2 * NSUB * CHUNK),
                          WINC)
    e_deg = 2 * NSUB * CHUNK * deg_chunks

    i32 = jnp.int32
    src_a = jnp.concatenate([src, jnp.zeros((e_agg - e,), i32)])
    src_a = src_a.reshape(NSUB, agg_chunks, GCH)
    dst_a = jnp.concatenate([dst, jnp.full((e_agg - e,), N, i32)])
    dst_a = dst_a.reshape(NSUB, agg_chunks, GCH)
    dst_d = jnp.concatenate([dst, jnp.full((e_deg - e,), N, i32)])
    dst_d = dst_d.reshape(2 * NSUB, deg_chunks, CHUNK)

    zrows = jnp.zeros((ACC_ROWS // NSUB, H), jnp.float32)
    ones_rows = jnp.ones((CHUNK, H), jnp.float32)

    deg0, deg1 = _make_deg_call(deg_chunks)(dst_d, ones_rows, zrows)
    hs0, hs1, dinv = _l1_call(x, W1, deg0, deg1)

    agg = _make_agg_call(agg_chunks)
    for b_prev, W in ((b1, W2), (b2, W3), (b3, W4), (b4, W5)):
        acc0, acc1 = agg(hs0, hs1, src_a, dst_a, zrows)
        hs0, hs1 = _mid_call(acc0, acc1, hs0, hs1, dinv,
                             b_prev.reshape(2, H), W)
    acc0, acc1 = agg(hs0, hs1, src_a, dst_a, zrows)
    return _fin_call(acc0, acc1, hs0, hs1, dinv, b5.reshape(2, H))


# speedup vs baseline: 6.7366x; 1.0698x over previous
import functools

import jax
import jax.numpy as jnp
from jax import lax
from jax.experimental import pallas as pl
from jax.experimental.pallas import tpu as pltpu
from jax.experimental.pallas import tpu_sc as plsc

N = 10000
D = 256
H = 128
CHUNK = 128
NSUB = 16
ACC_ROWS = 10240
NPT = 624
BN = 1000


def _ceil_to(v, m):
    return -(-v // m) * m


@functools.cache
def _mesh():
    return plsc.VectorSubcoreMesh(core_axis_name="c", subcore_axis_name="s",
                                  num_cores=2, num_subcores=NSUB)


def _copy_out(acc, out, s):
    @pl.when(s < NSUB - 1)
    def _():
        o = pl.multiple_of(s * NPT, 8)
        pltpu.sync_copy(acc.at[pl.ds(o, NPT)], out.at[pl.ds(o, NPT)])

    @pl.when(s == NSUB - 1)
    def _():
        last = N - (NSUB - 1) * NPT
        pltpu.sync_copy(acc.at[pl.ds((NSUB - 1) * NPT, last)],
                        out.at[pl.ds((NSUB - 1) * NPT, last)])


def _deg_body(dst_hbm, ones_hbm, z_hbm, out0, out1, accd, idx_d, ones_v, ssem):
    c = lax.axis_index("c")
    s = lax.axis_index("s")
    w = c * NSUB + s
    zr = ACC_ROWS // NSUB
    pltpu.sync_copy(z_hbm, accd.at[pl.ds(pl.multiple_of(s * zr, 8), zr)])
    pltpu.sync_copy(ones_hbm, ones_v)
    plsc.subcore_barrier()
    nchunks = dst_hbm.shape[1]

    def window(wi, carry):
        wo = pl.multiple_of(wi * WINC, 8)
        pltpu.sync_copy(dst_hbm.at[w, pl.ds(wo, WINC)], idx_d)
        ds = [pltpu.async_copy(ones_v, accd.at[idx_d.at[j]], ssem, add=True)
              for j in range(WINC)]
        for d in ds:
            d.wait()
        return carry

    lax.fori_loop(0, nchunks // WINC, window, 0)
    plsc.subcore_barrier()

    @pl.when(c == 0)
    def _():
        _copy_out(accd, out0, s)

    @pl.when(c == 1)
    def _():
        _copy_out(accd, out1, s)


GCH = 128
NBUF = 2
WINC = 16


def _agg_body(hs0, hs1, src_hbm, dst_hbm, z_hbm, out0, out1,
              acc, idx_s, idx_d, buf0, buf1,
              gsem0, gsem1, ssem0, ssem1):
    bufs = (buf0, buf1)
    gsems = (gsem0, gsem1)
    ssems = (ssem0, ssem1)
    c = lax.axis_index("c")
    s = lax.axis_index("s")
    zr = ACC_ROWS // NSUB
    pltpu.sync_copy(z_hbm, acc.at[pl.ds(pl.multiple_of(s * zr, 8), zr)])
    plsc.subcore_barrier()
    nchunks = src_hbm.shape[1]

    def run(hs):
        def window(w, carry):
            wo = pl.multiple_of(w * WINC, 8)
            pltpu.sync_copy(src_hbm.at[s, pl.ds(wo, WINC)], idx_s)
            pltpu.sync_copy(dst_hbm.at[s, pl.ds(wo, WINC)], idx_d)
            g = pltpu.async_copy(hs.at[idx_s.at[0]], bufs[0], gsems[0])
            for j in range(WINC):
                b = j % 2
                gn = None
                if j + 1 < WINC:
                    gn = pltpu.async_copy(hs.at[idx_s.at[j + 1]],
                                          bufs[1 - b], gsems[1 - b])
                g.wait()
                pltpu.sync_copy(bufs[b], acc.at[idx_d.at[j]], add=True)
                g = gn
            return carry

        lax.fori_loop(0, nchunks // WINC, window, 0)

    @pl.when(c == 0)
    def _():
        run(hs0)

    @pl.when(c == 1)
    def _():
        run(hs1)

    plsc.subcore_barrier()

    @pl.when(c == 0)
    def _():
        _copy_out(acc, out0, s)

    @pl.when(c == 1)
    def _():
        _copy_out(acc, out1, s)


def _make_deg_call(nchunks):
    return pl.kernel(
        _deg_body,
        out_type=[jax.ShapeDtypeStruct((N, H), jnp.float32),
                  jax.ShapeDtypeStruct((N, H), jnp.float32)],
        mesh=_mesh(),
        scratch_types=[
            pltpu.VMEM_SHARED((ACC_ROWS, H), jnp.float32),
            pltpu.VMEM((WINC, CHUNK), jnp.int32),
            pltpu.VMEM((CHUNK, H), jnp.float32),
            pltpu.SemaphoreType.DMA,
        ],
    )


def _make_agg_call(nchunks):
    return pl.kernel(
        _agg_body,
        out_type=[jax.ShapeDtypeStruct((N, H), jnp.float32),
                  jax.ShapeDtypeStruct((N, H), jnp.float32)],
        mesh=_mesh(),
        scratch_types=[
            pltpu.VMEM_SHARED((ACC_ROWS, H), jnp.float32),
            pltpu.VMEM((WINC, GCH), jnp.int32),
            pltpu.VMEM((WINC, GCH), jnp.int32),
            pltpu.VMEM((GCH, H), jnp.float32),
            pltpu.VMEM((GCH, H), jnp.float32),
        ] + [pltpu.SemaphoreType.DMA] * 4,
    )


def _l1_body(x_ref, w_ref, d0_ref, d1_ref, hs0_ref, hs1_ref, dinv_ref):
    deg = d0_ref[:, :1] + d1_ref[:, :1] + 1.0
    dinv = lax.rsqrt(deg)
    h = jnp.dot(x_ref[...], w_ref[...], preferred_element_type=jnp.float32)
    hs = h * dinv
    hs0_ref[...] = hs[:, :H]
    hs1_ref[...] = hs[:, H:]
    dinv_ref[...] = jnp.broadcast_to(dinv, (BN, H))


def _mid_body(a0_ref, a1_ref, h0_ref, h1_ref, dinv_ref, b_ref, w_ref,
              o0_ref, o1_ref):
    dinv = dinv_ref[...]
    a0 = jnp.maximum(dinv * (a0_ref[...] + h0_ref[...]) + b_ref[0:1, :], 0.0)
    a1 = jnp.maximum(dinv * (a1_ref[...] + h1_ref[...]) + b_ref[1:2, :], 0.0)
    a = jnp.concatenate([a0, a1], axis=1)
    h = jnp.dot(a, w_ref[...], preferred_element_type=jnp.float32)
    hs = h * dinv[:, :1]
    o0_ref[...] = hs[:, :H]
    o1_ref[...] = hs[:, H:]


def _fin_body(a0_ref, a1_ref, h0_ref, h1_ref, dinv_ref, b_ref, out_ref):
    dinv = dinv_ref[...]
    v0 = dinv * (a0_ref[...] + h0_ref[...]) + b_ref[0:1, :]
    v1 = dinv * (a1_ref[...] + h1_ref[...]) + b_ref[1:2, :]
    v = jnp.concatenate([v0, v1], axis=1)
    m = jnp.max(v, axis=1, keepdims=True)
    lse = jnp.log(jnp.sum(jnp.exp(v - m), axis=1, keepdims=True)) + m
    out_ref[...] = v - lse


def _rows(shape):
    return pl.BlockSpec(shape, lambda i: (i,) + (0,) * (len(shape) - 1))


def _full(shape):
    return pl.BlockSpec(shape, lambda i: (0,) * len(shape))


_GRID = N // BN

_l1_call = pl.pallas_call(
    _l1_body,
    grid=(_GRID,),
    in_specs=[_rows((BN, D)), _full((D, D)), _rows((BN, H)), _rows((BN, H))],
    out_specs=[_rows((BN, H)), _rows((BN, H)), _rows((BN, H))],
    out_shape=[jax.ShapeDtypeStruct((N, H), jnp.float32),
               jax.ShapeDtypeStruct((N, H), jnp.float32),
               jax.ShapeDtypeStruct((N, H), jnp.float32)],
)

_mid_call = pl.pallas_call(
    _mid_body,
    grid=(_GRID,),
    in_specs=[_rows((BN, H))] * 5 + [_full((2, H)), _full((D, D))],
    out_specs=[_rows((BN, H)), _rows((BN, H))],
    out_shape=[jax.ShapeDtypeStruct((N, H), jnp.float32),
               jax.ShapeDtypeStruct((N, H), jnp.float32)],
)

_fin_call = pl.pallas_call(
    _fin_body,
    grid=(_GRID,),
    in_specs=[_rows((BN, H))] * 5 + [_full((2, H))],
    out_specs=_rows((BN, D)),
    out_shape=jax.ShapeDtypeStruct((N, D), jnp.float32),
)


def kernel(x, edge_index, W1, b1, W2, b2, W3, b3, W4, b4, W5, b5):
    src = edge_index[0]
    dst = edge_index[1]
    e = src.shape[0]

    agg_chunks = _ceil_to(_ceil_to(e, NSUB * GCH) // (NSUB * GCH), WINC)
    e_agg = NSUB * GCH * agg_chunks
    deg_chunks = _ceil_to(_ceil_to(e, 2 * NSUB * CHUNK) // (2 * NSUB * CHUNK),
                          WINC)
    e_deg = 2 * NSUB * CHUNK * deg_chunks

    i32 = jnp.int32
    src_a = jnp.concatenate([src, jnp.zeros((e_agg - e,), i32)])
    src_a = src_a.reshape(NSUB, agg_chunks, GCH)
    dst_a = jnp.concatenate([dst, jnp.full((e_agg - e,), N, i32)])
    dst_a = dst_a.reshape(NSUB, agg_chunks, GCH)
    dst_d = jnp.concatenate([dst, jnp.full((e_deg - e,), N, i32)])
    dst_d = dst_d.reshape(2 * NSUB, deg_chunks, CHUNK)

    zrows = jnp.zeros((ACC_ROWS // NSUB, H), jnp.float32)
    ones_rows = jnp.ones((CHUNK, H), jnp.float32)

    deg0, deg1 = _make_deg_call(deg_chunks)(dst_d, ones_rows, zrows)
    hs0, hs1, dinv = _l1_call(x, W1, deg0, deg1)

    agg = _make_agg_call(agg_chunks)
    for b_prev, W in ((b1, W2), (b2, W3), (b3, W4), (b4, W5)):
        acc0, acc1 = agg(hs0, hs1, src_a, dst_a, zrows)
        hs0, hs1 = _mid_call(acc0, acc1, hs0, hs1, dinv,
                             b_prev.reshape(2, H), W)
    acc0, acc1 = agg(hs0, hs1, src_a, dst_a, zrows)
    return _fin_call(acc0, acc1, hs0, hs1, dinv, b5.reshape(2, H))


# speedup vs baseline: 6.8480x; 1.0165x over previous
import functools

import jax
import jax.numpy as jnp
from jax import lax
from jax.experimental import pallas as pl
from jax.experimental.pallas import tpu as pltpu
from jax.experimental.pallas import tpu_sc as plsc

N = 10000
D = 256
H = 128
CHUNK = 128
NSUB = 16
ACC_ROWS = 10240
NPT = 624
BN = 1000


def _ceil_to(v, m):
    return -(-v // m) * m


@functools.cache
def _mesh():
    return plsc.VectorSubcoreMesh(core_axis_name="c", subcore_axis_name="s",
                                  num_cores=2, num_subcores=NSUB)


def _copy_out(acc, out, s):
    @pl.when(s < NSUB - 1)
    def _():
        o = pl.multiple_of(s * NPT, 8)
        pltpu.sync_copy(acc.at[pl.ds(o, NPT)], out.at[pl.ds(o, NPT)])

    @pl.when(s == NSUB - 1)
    def _():
        last = N - (NSUB - 1) * NPT
        pltpu.sync_copy(acc.at[pl.ds((NSUB - 1) * NPT, last)],
                        out.at[pl.ds((NSUB - 1) * NPT, last)])


def _deg_body(dst_hbm, ones_hbm, z_hbm, out0, out1, accd, idx_d, ones_v, ssem):
    c = lax.axis_index("c")
    s = lax.axis_index("s")
    w = c * NSUB + s
    zr = ACC_ROWS // NSUB
    pltpu.sync_copy(z_hbm, accd.at[pl.ds(pl.multiple_of(s * zr, 8), zr)])
    pltpu.sync_copy(ones_hbm, ones_v)
    plsc.subcore_barrier()
    nchunks = dst_hbm.shape[1]

    def window(wi, carry):
        wo = pl.multiple_of(wi * WINC, 8)
        pltpu.sync_copy(dst_hbm.at[w, pl.ds(wo, WINC)], idx_d)
        ds = [pltpu.async_copy(ones_v, accd.at[idx_d.at[j]], ssem, add=True)
              for j in range(WINC)]
        for d in ds:
            d.wait()
        return carry

    lax.fori_loop(0, nchunks // WINC, window, 0)
    plsc.subcore_barrier()

    @pl.when(c == 0)
    def _():
        _copy_out(accd, out0, s)

    @pl.when(c == 1)
    def _():
        _copy_out(accd, out1, s)


GCH = 128
NBUF = 2
WINC = 16


def _agg_body(hs0, hs1, src_hbm, dst_hbm, z_hbm, out0, out1,
              acc, ixa_s, ixa_d, ixb_s, ixb_d, buf0, buf1,
              gsem0, gsem1, sta_s, sta_d, stb_s, stb_d):
    bufs = (buf0, buf1)
    gsems = (gsem0, gsem1)
    c = lax.axis_index("c")
    s = lax.axis_index("s")
    zr = ACC_ROWS // NSUB
    pltpu.sync_copy(z_hbm, acc.at[pl.ds(pl.multiple_of(s * zr, 8), zr)])
    nchunks = src_hbm.shape[1]
    nwin = nchunks // WINC
    ix = ((ixa_s, ixa_d, sta_s, sta_d), (ixb_s, ixb_d, stb_s, stb_d))

    def stage(w, slot):
        wo = pl.multiple_of(w * WINC, 8)
        i_s, i_d, m_s, m_d = ix[slot]
        pltpu.async_copy(src_hbm.at[s, pl.ds(wo, WINC)], i_s, m_s)
        pltpu.async_copy(dst_hbm.at[s, pl.ds(wo, WINC)], i_d, m_d)

    def stage_wait(slot):
        i_s, i_d, m_s, m_d = ix[slot]
        pltpu.make_async_copy(src_hbm.at[s, pl.ds(0, WINC)], i_s, m_s).wait()
        pltpu.make_async_copy(dst_hbm.at[s, pl.ds(0, WINC)], i_d, m_d).wait()

    stage(0, 0)
    plsc.subcore_barrier()

    def run(hs):
        def process(w, slot):
            nxt = lax.rem(w + 1, nwin)
            stage(nxt, 1 - slot)
            stage_wait(slot)
            idx_s, idx_d = ix[slot][0], ix[slot][1]
            g = pltpu.async_copy(hs.at[idx_s.at[0]], bufs[0], gsems[0])
            for j in range(WINC):
                b = j % 2
                gn = None
                if j + 1 < WINC:
                    gn = pltpu.async_copy(hs.at[idx_s.at[j + 1]],
                                          bufs[1 - b], gsems[1 - b])
                g.wait()
                pltpu.sync_copy(bufs[b], acc.at[idx_d.at[j]], add=True)
                g = gn

        def wpair(i, carry):
            process(2 * i, 0)
            process(2 * i + 1, 1)
            return carry

        lax.fori_loop(0, nwin // 2, wpair, 0)
        if nwin % 2:
            process(nwin - 1, 0)
            stage_wait(1)
        else:
            stage_wait(0)

    @pl.when(c == 0)
    def _():
        run(hs0)

    @pl.when(c == 1)
    def _():
        run(hs1)

    plsc.subcore_barrier()

    @pl.when(c == 0)
    def _():
        _copy_out(acc, out0, s)

    @pl.when(c == 1)
    def _():
        _copy_out(acc, out1, s)


def _make_deg_call(nchunks):
    return pl.kernel(
        _deg_body,
        out_type=[jax.ShapeDtypeStruct((N, H), jnp.float32),
                  jax.ShapeDtypeStruct((N, H), jnp.float32)],
        mesh=_mesh(),
        scratch_types=[
            pltpu.VMEM_SHARED((ACC_ROWS, H), jnp.float32),
            pltpu.VMEM((WINC, CHUNK), jnp.int32),
            pltpu.VMEM((CHUNK, H), jnp.float32),
            pltpu.SemaphoreType.DMA,
        ],
    )


def _make_agg_call(nchunks):
    return pl.kernel(
        _agg_body,
        out_type=[jax.ShapeDtypeStruct((N, H), jnp.float32),
                  jax.ShapeDtypeStruct((N, H), jnp.float32)],
        mesh=_mesh(),
        scratch_types=[
            pltpu.VMEM_SHARED((ACC_ROWS, H), jnp.float32),
            pltpu.VMEM((WINC, GCH), jnp.int32),
            pltpu.VMEM((WINC, GCH), jnp.int32),
            pltpu.VMEM((WINC, GCH), jnp.int32),
            pltpu.VMEM((WINC, GCH), jnp.int32),
            pltpu.VMEM((GCH, H), jnp.float32),
            pltpu.VMEM((GCH, H), jnp.float32),
        ] + [pltpu.SemaphoreType.DMA] * 6,
    )


def _l1_body(x_ref, w_ref, d0_ref, d1_ref, hs0_ref, hs1_ref, dinv_ref):
    deg = d0_ref[:, :1] + d1_ref[:, :1] + 1.0
    dinv = lax.rsqrt(deg)
    h = jnp.dot(x_ref[...], w_ref[...], preferred_element_type=jnp.float32)
    hs = h * dinv
    hs0_ref[...] = hs[:, :H]
    hs1_ref[...] = hs[:, H:]
    dinv_ref[...] = jnp.broadcast_to(dinv, (BN, H))


def _mid_body(a0_ref, a1_ref, h0_ref, h1_ref, dinv_ref, b_ref, w_ref,
              o0_ref, o1_ref):
    dinv = dinv_ref[...]
    a0 = jnp.maximum(dinv * (a0_ref[...] + h0_ref[...]) + b_ref[0:1, :], 0.0)
    a1 = jnp.maximum(dinv * (a1_ref[...] + h1_ref[...]) + b_ref[1:2, :], 0.0)
    a = jnp.concatenate([a0, a1], axis=1)
    h = jnp.dot(a, w_ref[...], preferred_element_type=jnp.float32)
    hs = h * dinv[:, :1]
    o0_ref[...] = hs[:, :H]
    o1_ref[...] = hs[:, H:]


def _fin_body(a0_ref, a1_ref, h0_ref, h1_ref, dinv_ref, b_ref, out_ref):
    dinv = dinv_ref[...]
    v0 = dinv * (a0_ref[...] + h0_ref[...]) + b_ref[0:1, :]
    v1 = dinv * (a1_ref[...] + h1_ref[...]) + b_ref[1:2, :]
    v = jnp.concatenate([v0, v1], axis=1)
    m = jnp.max(v, axis=1, keepdims=True)
    lse = jnp.log(jnp.sum(jnp.exp(v - m), axis=1, keepdims=True)) + m
    out_ref[...] = v - lse


def _rows(shape):
    return pl.BlockSpec(shape, lambda i: (i,) + (0,) * (len(shape) - 1))


def _full(shape):
    return pl.BlockSpec(shape, lambda i: (0,) * len(shape))


_GRID = N // BN

_l1_call = pl.pallas_call(
    _l1_body,
    grid=(_GRID,),
    in_specs=[_rows((BN, D)), _full((D, D)), _rows((BN, H)), _rows((BN, H))],
    out_specs=[_rows((BN, H)), _rows((BN, H)), _rows((BN, H))],
    out_shape=[jax.ShapeDtypeStruct((N, H), jnp.float32),
               jax.ShapeDtypeStruct((N, H), jnp.float32),
               jax.ShapeDtypeStruct((N, H), jnp.float32)],
)

_mid_call = pl.pallas_call(
    _mid_body,
    grid=(_GRID,),
    in_specs=[_rows((BN, H))] * 5 + [_full((2, H)), _full((D, D))],
    out_specs=[_rows((BN, H)), _rows((BN, H))],
    out_shape=[jax.ShapeDtypeStruct((N, H), jnp.float32),
               jax.ShapeDtypeStruct((N, H), jnp.float32)],
)

_fin_call = pl.pallas_call(
    _fin_body,
    grid=(_GRID,),
    in_specs=[_rows((BN, H))] * 5 + [_full((2, H))],
    out_specs=_rows((BN, D)),
    out_shape=jax.ShapeDtypeStruct((N, D), jnp.float32),
)


def kernel(x, edge_index, W1, b1, W2, b2, W3, b3, W4, b4, W5, b5):
    src = edge_index[0]
    dst = edge_index[1]
    e = src.shape[0]

    agg_chunks = _ceil_to(_ceil_to(e, NSUB * GCH) // (NSUB * GCH), WINC)
    e_agg = NSUB * GCH * agg_chunks
    deg_chunks = _ceil_to(_ceil_to(e, 2 * NSUB * CHUNK) // (2 * NSUB * CHUNK),
                          WINC)
    e_deg = 2 * NSUB * CHUNK * deg_chunks

    i32 = jnp.int32
    src_a = jnp.concatenate([src, jnp.zeros((e_agg - e,), i32)])
    src_a = src_a.reshape(NSUB, agg_chunks, GCH)
    dst_a = jnp.concatenate([dst, jnp.full((e_agg - e,), N, i32)])
    dst_a = dst_a.reshape(NSUB, agg_chunks, GCH)
    dst_d = jnp.concatenate([dst, jnp.full((e_deg - e,), N, i32)])
    dst_d = dst_d.reshape(2 * NSUB, deg_chunks, CHUNK)

    zrows = jnp.zeros((ACC_ROWS // NSUB, H), jnp.float32)
    ones_rows = jnp.ones((CHUNK, H), jnp.float32)

    deg0, deg1 = _make_deg_call(deg_chunks)(dst_d, ones_rows, zrows)
    hs0, hs1, dinv = _l1_call(x, W1, deg0, deg1)

    agg = _make_agg_call(agg_chunks)
    for b_prev, W in ((b1, W2), (b2, W3), (b3, W4), (b4, W5)):
        acc0, acc1 = agg(hs0, hs1, src_a, dst_a, zrows)
        hs0, hs1 = _mid_call(acc0, acc1, hs0, hs1, dinv,
                             b_prev.reshape(2, H), W)
    acc0, acc1 = agg(hs0, hs1, src_a, dst_a, zrows)
    return _fin_call(acc0, acc1, hs0, hs1, dinv, b5.reshape(2, H))
